# baseline (speedup 1.0000x reference)
import functools

import jax
import jax.numpy as jnp
from jax import lax
from jax.experimental import pallas as pl
from jax.experimental.pallas import tpu as pltpu
from jax.experimental.pallas import tpu_sc as plsc

B = 16384
N_SPARSE = 26
N_DENSE = 13
N_COLS = N_SPARSE + N_DENSE
VOCAB = 1000000
VOCAB_PAD = 1000064
L = 16


def _build_sc_kernel():
    info = plsc.get_sparse_core_info()
    nc, ns = info.num_cores, info.num_subcores
    nw = nc * ns
    rows_w = B // nw
    blocks_w = rows_w // L
    n_idx = N_SPARSE * rows_w
    idx_rows = n_idx // 128

    mesh = plsc.VectorSubcoreMesh(core_axis_name="c", subcore_axis_name="s")

    @functools.partial(
        pl.kernel,
        out_type=jax.ShapeDtypeStruct((B,), jnp.float32),
        mesh=mesh,
        compiler_params=pltpu.CompilerParams(use_tc_tiling_on_sc=False),
        scratch_types=[
            pltpu.VMEM((N_COLS, rows_w), jnp.float32),
            pltpu.VMEM((n_idx,), jnp.int32),
            pltpu.VMEM((n_idx,), jnp.float32),
            pltpu.VMEM((N_DENSE * L,), jnp.float32),
            pltpu.VMEM((rows_w,), jnp.float32),
            pltpu.SemaphoreType.DMA,
        ],
    )
    def sc_kernel(xt_hbm, table_hbm, wb_hbm, out_hbm, xt_v, idx_v, gath_v,
                  wb_v, out_v, sem):
        wid = lax.axis_index("s") * nc + lax.axis_index("c")
        base_row = wid * rows_w

        pltpu.sync_copy(xt_hbm.at[:, pl.ds(base_row, rows_w)], xt_v)
        pltpu.sync_copy(wb_hbm, wb_v)

        def idx_body(t, _):
            f = t // blocks_w
            blk = t % blocks_w
            xv = xt_v[f, pl.ds(blk * L, L)]
            idx_v[pl.ds(t * L, L)] = xv.astype(jnp.int32)
            return 0

        lax.fori_loop(0, N_SPARSE * blocks_w, idx_body, 0)

        for f in range(N_SPARSE):
            pltpu.async_copy(
                table_hbm.at[f].at[idx_v.at[pl.ds(f * rows_w, rows_w)]],
                gath_v.at[pl.ds(f * rows_w, rows_w)], sem)
        for _ in range(N_SPARSE):
            pltpu.make_async_copy(
                table_hbm.at[0].at[idx_v.at[pl.ds(0, rows_w)]],
                gath_v.at[pl.ds(0, rows_w)], sem).wait()

        def blk_body(blk, _):
            def red_body(f, acc):
                g = gath_v[pl.ds(f * rows_w + blk * L, L)]
                return acc + g

            acc = lax.fori_loop(0, N_SPARSE, red_body,
                                jnp.zeros((L,), jnp.float32))

            def dense_body(d, dacc):
                xv = xt_v[N_SPARSE + d, pl.ds(blk * L, L)]
                wv = wb_v[pl.ds(d * L, L)]
                return dacc + xv * wv

            acc = lax.fori_loop(0, N_DENSE, dense_body, acc)
            out_v[pl.ds(blk * L, L)] = 1.0 / (1.0 + jnp.exp(-acc))
            return 0

        lax.fori_loop(0, blocks_w, blk_body, 0)

        pltpu.sync_copy(out_v, out_hbm.at[pl.ds(base_row, rows_w)])

    return sc_kernel


def kernel(X, emb_tables, dense_weight):
    xt = X.T
    table2 = emb_tables.reshape(N_SPARSE, VOCAB)
    w_bcast = jnp.repeat(dense_weight.reshape(-1), L)
    out = _build_sc_kernel()(xt, table2, w_bcast)
    return out[:, None]


# speedup vs baseline: 5.0088x; 5.0088x over previous
import functools

import jax
import jax.numpy as jnp
from jax import lax
from jax.experimental import pallas as pl
from jax.experimental.pallas import tpu as pltpu
from jax.experimental.pallas import tpu_sc as plsc

B = 16384
N_SPARSE = 26
N_DENSE = 13
N_COLS = N_SPARSE + N_DENSE
VOCAB = 1000000
VOCAB_PAD = 1000064
L = 16


def _build_sc_kernel():
    info = plsc.get_sparse_core_info()
    nc, ns = info.num_cores, info.num_subcores
    nw = nc * ns
    rows_w = B // nw
    blocks_w = rows_w // L
    n_idx = N_SPARSE * rows_w
    idx_rows = n_idx // 128

    mesh = plsc.VectorSubcoreMesh(core_axis_name="c", subcore_axis_name="s")

    @functools.partial(
        pl.kernel,
        out_type=jax.ShapeDtypeStruct((B,), jnp.float32),
        mesh=mesh,
        compiler_params=pltpu.CompilerParams(use_tc_tiling_on_sc=False),
        scratch_types=[
            pltpu.VMEM((N_COLS, rows_w), jnp.float32),
            pltpu.VMEM((n_idx,), jnp.int32),
            pltpu.VMEM((n_idx,), jnp.float32),
            pltpu.VMEM((N_DENSE * L,), jnp.float32),
            pltpu.VMEM((rows_w,), jnp.float32),
            pltpu.SemaphoreType.DMA,
        ],
    )
    def sc_kernel(xt_hbm, *rest):
        table_refs = rest[:N_SPARSE]
        (wb_hbm, out_hbm, xt_v, idx_v, gath_v, wb_v, out_v, sem) = \
            rest[N_SPARSE:]
        wid = lax.axis_index("s") * nc + lax.axis_index("c")
        base_row = wid * rows_w

        pltpu.sync_copy(xt_hbm.at[:, pl.ds(base_row, rows_w)], xt_v)
        pltpu.sync_copy(wb_hbm, wb_v)

        def idx_body(t, _):
            f = t // blocks_w
            blk = t % blocks_w
            xv = xt_v[f, pl.ds(blk * L, L)]
            idx_v[pl.ds(t * L, L)] = xv.astype(jnp.int32)
            return 0

        lax.fori_loop(0, N_SPARSE * blocks_w, idx_body, 0)

        for f in range(N_SPARSE):
            pltpu.async_copy(
                table_refs[f].at[idx_v.at[pl.ds(f * rows_w, rows_w)]],
                gath_v.at[pl.ds(f * rows_w, rows_w)], sem)
        for f in range(N_SPARSE):
            pltpu.make_async_copy(
                table_refs[0].at[idx_v.at[pl.ds(0, rows_w)]],
                gath_v.at[pl.ds(0, rows_w)], sem).wait()

        def blk_body(blk, _):
            def red_body(f, acc):
                g = gath_v[pl.ds(f * rows_w + blk * L, L)]
                return acc + g

            acc = lax.fori_loop(0, N_SPARSE, red_body,
                                jnp.zeros((L,), jnp.float32))

            def dense_body(d, dacc):
                xv = xt_v[N_SPARSE + d, pl.ds(blk * L, L)]
                wv = wb_v[pl.ds(d * L, L)]
                return dacc + xv * wv

            acc = lax.fori_loop(0, N_DENSE, dense_body, acc)
            out_v[pl.ds(blk * L, L)] = 1.0 / (1.0 + jnp.exp(-acc))
            return 0

        lax.fori_loop(0, blocks_w, blk_body, 0)

        pltpu.sync_copy(out_v, out_hbm.at[pl.ds(base_row, rows_w)])

    return sc_kernel


def kernel(X, emb_tables, dense_weight):
    xt = X.T
    table_rows = [emb_tables[f, :, 0] for f in range(N_SPARSE)]
    w_bcast = jnp.repeat(dense_weight.reshape(-1), L)
    out = _build_sc_kernel()(xt, *table_rows, w_bcast)
    return out[:, None]


# speedup vs baseline: 15.5513x; 3.1048x over previous
import functools

import jax
import jax.numpy as jnp
from jax import lax
from jax.experimental import pallas as pl
from jax.experimental.pallas import tpu as pltpu
from jax.experimental.pallas import tpu_sc as plsc

B = 16384
N_SPARSE = 26
N_DENSE = 13
N_COLS = N_SPARSE + N_DENSE
VOCAB = 1000000
L = 16

SEGS = 8
SEG_W = VOCAB // SEGS


FIELD_STRIDE = 1 << 20
MAIN_W = (VOCAB // 128) * 128
TAIL = VOCAB - MAIN_W
PAD_W = FIELD_STRIDE - MAIN_W


def _untile_body(t_hbm, tail_ref, o_hbm, buf_v, tz_v, sem):
    f = pl.program_id(0)
    tz_v[...] = jnp.zeros((PAD_W,), jnp.float32)
    c1 = pltpu.make_async_copy(t_hbm.at[f, 0, pl.ds(0, MAIN_W)], buf_v, sem)
    c1.start()
    tz_v[pl.ds(0, TAIL)] = tail_ref[f, :]
    c1.wait()
    c2 = pltpu.make_async_copy(
        buf_v, o_hbm.at[pl.ds(f * FIELD_STRIDE, MAIN_W)], sem)
    c2.start()
    c4 = pltpu.make_async_copy(
        tz_v, o_hbm.at[pl.ds(f * FIELD_STRIDE + MAIN_W, PAD_W)], sem)
    c4.start()
    c2.wait()
    c4.wait()


def _tc_untile(emb_tables):
    return pl.pallas_call(
        _untile_body,
        grid=(N_SPARSE,),
        in_specs=[
            pl.BlockSpec(memory_space=pltpu.HBM),
            pl.BlockSpec((N_SPARSE, TAIL), lambda f: (0, 0)),
        ],
        out_specs=pl.BlockSpec(memory_space=pltpu.HBM),
        out_shape=jax.ShapeDtypeStruct((N_SPARSE * FIELD_STRIDE,),
                                       jnp.float32),
        scratch_shapes=[
            pltpu.VMEM((MAIN_W,), jnp.float32),
            pltpu.VMEM((PAD_W,), jnp.float32),
            pltpu.SemaphoreType.DMA,
        ],
    )(emb_tables.transpose(0, 2, 1), emb_tables[:, MAIN_W:, 0])


def _build_lookup_kernel():
    info = plsc.get_sparse_core_info()
    nc, ns = info.num_cores, info.num_subcores
    nw = nc * ns
    rows_w = B // nw
    blocks_w = rows_w // L
    n_idx = N_SPARSE * rows_w

    mesh = plsc.VectorSubcoreMesh(core_axis_name="c", subcore_axis_name="s")

    @functools.partial(
        pl.kernel,
        out_type=jax.ShapeDtypeStruct((B,), jnp.float32),
        mesh=mesh,
        scratch_types=[
            pltpu.VMEM((N_COLS, rows_w), jnp.float32),
            pltpu.VMEM((n_idx,), jnp.int32),
            pltpu.VMEM((n_idx,), jnp.float32),
            pltpu.VMEM((N_DENSE * L,), jnp.float32),
            pltpu.VMEM((rows_w,), jnp.float32),
            pltpu.SemaphoreType.DMA,
        ],
    )
    def lookup_kernel(xt_hbm, flat_hbm, wb_hbm, out_hbm, xt_v, idx_v, gath_v,
                      wb_v, out_v, sem):
        wid = lax.axis_index("s") * nc + lax.axis_index("c")
        base_row = wid * rows_w

        pltpu.sync_copy(xt_hbm.at[:, pl.ds(base_row, rows_w)], xt_v)
        pltpu.sync_copy(wb_hbm, wb_v)

        def idx_body(t, _):
            f = t // blocks_w
            blk = t % blocks_w
            xv = xt_v[f, pl.ds(blk * L, L)]
            idx_v[pl.ds(t * L, L)] = xv.astype(jnp.int32) + f * FIELD_STRIDE
            return 0

        lax.fori_loop(0, N_SPARSE * blocks_w, idx_body, 0)

        pltpu.async_copy(flat_hbm.at[idx_v], gath_v, sem).wait()

        def blk_body(blk, _):
            def red_body(f, acc):
                g = gath_v[pl.ds(f * rows_w + blk * L, L)]
                return acc + g

            acc = lax.fori_loop(0, N_SPARSE, red_body,
                                jnp.zeros((L,), jnp.float32))

            def dense_body(d, dacc):
                xv = xt_v[N_SPARSE + d, pl.ds(blk * L, L)]
                wv = wb_v[pl.ds(d * L, L)]
                return dacc + xv * wv

            acc = lax.fori_loop(0, N_DENSE, dense_body, acc)
            out_v[pl.ds(blk * L, L)] = 1.0 / (1.0 + jnp.exp(-acc))
            return 0

        lax.fori_loop(0, blocks_w, blk_body, 0)

        pltpu.sync_copy(out_v, out_hbm.at[pl.ds(base_row, rows_w)])

    return lookup_kernel


def kernel(X, emb_tables, dense_weight):
    xt = X.T
    w_bcast = jnp.repeat(dense_weight.reshape(-1), L)
    flat = _tc_untile(emb_tables)
    out = _build_lookup_kernel()(xt, flat, w_bcast)
    return out[:, None]


# speedup vs baseline: 19.4363x; 1.2498x over previous
import functools

import jax
import jax.numpy as jnp
from jax import lax
from jax.experimental import pallas as pl
from jax.experimental.pallas import tpu as pltpu
from jax.experimental.pallas import tpu_sc as plsc

B = 16384
N_SPARSE = 26
N_DENSE = 13
N_COLS = N_SPARSE + N_DENSE
VOCAB = 1000000
L = 16

SEGS = 8
SEG_W = VOCAB // SEGS


FIELD_STRIDE = 1 << 20
MAIN_W = (VOCAB // 128) * 128
TAIL = VOCAB - MAIN_W
PAD_W = FIELD_STRIDE - MAIN_W


def _read(t_hbm, buf_v, g, rsem):
    return pltpu.make_async_copy(t_hbm.at[g, 0, pl.ds(0, MAIN_W)],
                                 buf_v.at[g % 2], rsem)


def _write(o_hbm, buf_v, g, wsem):
    return pltpu.make_async_copy(
        buf_v.at[g % 2], o_hbm.at[pl.ds(g * FIELD_STRIDE, MAIN_W)], wsem)


def _untile_body(t_hbm, tail_ref, o_hbm, buf_v, tz_v, rsem, wsem, psem):
    f = pl.program_id(0)

    @pl.when(f == 0)
    def _():
        tz_v[...] = jnp.zeros((PAD_W,), jnp.float32)
        _read(t_hbm, buf_v, 0, rsem).start()

    @pl.when(f > 0)
    def _():
        _write(o_hbm, buf_v, f - 1, wsem).wait()

    @pl.when(f < N_SPARSE - 1)
    def _():
        _read(t_hbm, buf_v, f + 1, rsem).start()

    tz_v[pl.ds(0, TAIL)] = tail_ref[f, :]
    _read(t_hbm, buf_v, f, rsem).wait()
    _write(o_hbm, buf_v, f, wsem).start()
    c4 = pltpu.make_async_copy(
        tz_v, o_hbm.at[pl.ds(f * FIELD_STRIDE + MAIN_W, PAD_W)], psem)
    c4.start()
    c4.wait()

    @pl.when(f == N_SPARSE - 1)
    def _():
        _write(o_hbm, buf_v, f, wsem).wait()


def _tc_untile(emb_tables):
    return pl.pallas_call(
        _untile_body,
        grid=(N_SPARSE,),
        in_specs=[
            pl.BlockSpec(memory_space=pltpu.HBM),
            pl.BlockSpec((N_SPARSE, TAIL), lambda f: (0, 0)),
        ],
        out_specs=pl.BlockSpec(memory_space=pltpu.HBM),
        out_shape=jax.ShapeDtypeStruct((N_SPARSE * FIELD_STRIDE,),
                                       jnp.float32),
        scratch_shapes=[
            pltpu.VMEM((2, MAIN_W), jnp.float32),
            pltpu.VMEM((PAD_W,), jnp.float32),
            pltpu.SemaphoreType.DMA,
            pltpu.SemaphoreType.DMA,
            pltpu.SemaphoreType.DMA,
        ],
    )(emb_tables.transpose(0, 2, 1), emb_tables[:, MAIN_W:, 0])


def _build_lookup_kernel():
    info = plsc.get_sparse_core_info()
    nc, ns = info.num_cores, info.num_subcores
    nw = nc * ns
    rows_w = B // nw
    blocks_w = rows_w // L
    n_idx = N_SPARSE * rows_w

    mesh = plsc.VectorSubcoreMesh(core_axis_name="c", subcore_axis_name="s")

    @functools.partial(
        pl.kernel,
        out_type=jax.ShapeDtypeStruct((B,), jnp.float32),
        mesh=mesh,
        scratch_types=[
            pltpu.VMEM((N_COLS, rows_w), jnp.float32),
            pltpu.VMEM((n_idx,), jnp.int32),
            pltpu.VMEM((n_idx,), jnp.float32),
            pltpu.VMEM((N_DENSE * L,), jnp.float32),
            pltpu.VMEM((rows_w,), jnp.float32),
            pltpu.SemaphoreType.DMA,
        ],
    )
    def lookup_kernel(xt_hbm, flat_hbm, wb_hbm, out_hbm, xt_v, idx_v, gath_v,
                      wb_v, out_v, sem):
        wid = lax.axis_index("s") * nc + lax.axis_index("c")
        base_row = wid * rows_w

        pltpu.sync_copy(xt_hbm.at[:, pl.ds(base_row, rows_w)], xt_v)
        pltpu.sync_copy(wb_hbm, wb_v)

        def idx_body(t, _):
            f = t // blocks_w
            blk = t % blocks_w
            xv = xt_v[f, pl.ds(blk * L, L)]
            idx_v[pl.ds(t * L, L)] = xv.astype(jnp.int32) + f * FIELD_STRIDE
            return 0

        lax.fori_loop(0, N_SPARSE * blocks_w, idx_body, 0)

        pltpu.async_copy(flat_hbm.at[idx_v], gath_v, sem).wait()

        def blk_body(blk, _):
            def red_body(f, acc):
                g = gath_v[pl.ds(f * rows_w + blk * L, L)]
                return acc + g

            acc = lax.fori_loop(0, N_SPARSE, red_body,
                                jnp.zeros((L,), jnp.float32))

            def dense_body(d, dacc):
                xv = xt_v[N_SPARSE + d, pl.ds(blk * L, L)]
                wv = wb_v[pl.ds(d * L, L)]
                return dacc + xv * wv

            acc = lax.fori_loop(0, N_DENSE, dense_body, acc)
            out_v[pl.ds(blk * L, L)] = 1.0 / (1.0 + jnp.exp(-acc))
            return 0

        lax.fori_loop(0, blocks_w, blk_body, 0)

        pltpu.sync_copy(out_v, out_hbm.at[pl.ds(base_row, rows_w)])

    return lookup_kernel


def kernel(X, emb_tables, dense_weight):
    xt = X.T
    w_bcast = jnp.repeat(dense_weight.reshape(-1), L)
    flat = _tc_untile(emb_tables)
    out = _build_lookup_kernel()(xt, flat, w_bcast)
    return out[:, None]


# speedup vs baseline: 20.8220x; 1.0713x over previous
import functools

import jax
import jax.numpy as jnp
from jax import lax
from jax.experimental import pallas as pl
from jax.experimental.pallas import tpu as pltpu
from jax.experimental.pallas import tpu_sc as plsc

B = 16384
N_SPARSE = 26
N_DENSE = 13
N_COLS = N_SPARSE + N_DENSE
VOCAB = 1000000
L = 16

SEGS = 8
SEG_W = VOCAB // SEGS


FIELD_STRIDE = 1 << 20
MAIN_W = (VOCAB // 128) * 128
TAIL = VOCAB - MAIN_W
PAD_W = FIELD_STRIDE - MAIN_W


NBUF = 3


def _read(t_hbm, buf_v, g, rsem):
    return pltpu.make_async_copy(t_hbm.at[g, 0, pl.ds(0, MAIN_W)],
                                 buf_v.at[g % NBUF], rsem)


def _write(o_hbm, buf_v, g, wsem):
    return pltpu.make_async_copy(
        buf_v.at[g % NBUF], o_hbm.at[pl.ds(g * FIELD_STRIDE, MAIN_W)], wsem)


def _pad_write(o_hbm, tz_v, g, psem):
    return pltpu.make_async_copy(
        tz_v.at[g % 2], o_hbm.at[pl.ds(g * FIELD_STRIDE + MAIN_W, PAD_W)],
        psem)


def _untile_body(t_hbm, tail_ref, o_hbm, buf_v, tz_v, rsem, wsem, psem):
    f = pl.program_id(0)

    @pl.when(f == 0)
    def _():
        tz_v[...] = jnp.zeros((2, PAD_W), jnp.float32)
        _read(t_hbm, buf_v, 0, rsem).start()
        _read(t_hbm, buf_v, 1, rsem).start()

    @pl.when(f > 1)
    def _():
        _pad_write(o_hbm, tz_v, f - 2, psem).wait()

    @pl.when(f % 2 == 0)
    def _():
        tz_v[0, pl.ds(0, TAIL)] = tail_ref[f, :]

    @pl.when(f % 2 == 1)
    def _():
        tz_v[1, pl.ds(0, TAIL)] = tail_ref[f, :]
    _read(t_hbm, buf_v, f, rsem).wait()
    _write(o_hbm, buf_v, f, wsem).start()
    _pad_write(o_hbm, tz_v, f, psem).start()

    @pl.when(f + 2 < N_SPARSE)
    def _():
        @pl.when(f >= 1)
        def _():
            _write(o_hbm, buf_v, f - 1, wsem).wait()

        _read(t_hbm, buf_v, f + 2, rsem).start()

    @pl.when(f == N_SPARSE - 1)
    def _():
        _write(o_hbm, buf_v, f - 2, wsem).wait()
        _write(o_hbm, buf_v, f - 1, wsem).wait()
        _write(o_hbm, buf_v, f, wsem).wait()
        _pad_write(o_hbm, tz_v, f - 1, psem).wait()
        _pad_write(o_hbm, tz_v, f, psem).wait()


def _tc_untile(emb_tables):
    return pl.pallas_call(
        _untile_body,
        grid=(N_SPARSE,),
        in_specs=[
            pl.BlockSpec(memory_space=pltpu.HBM),
            pl.BlockSpec((N_SPARSE, TAIL), lambda f: (0, 0)),
        ],
        out_specs=pl.BlockSpec(memory_space=pltpu.HBM),
        out_shape=jax.ShapeDtypeStruct((N_SPARSE * FIELD_STRIDE,),
                                       jnp.float32),
        scratch_shapes=[
            pltpu.VMEM((NBUF, MAIN_W), jnp.float32),
            pltpu.VMEM((2, PAD_W), jnp.float32),
            pltpu.SemaphoreType.DMA,
            pltpu.SemaphoreType.DMA,
            pltpu.SemaphoreType.DMA,
        ],
    )(emb_tables.transpose(0, 2, 1), emb_tables[:, MAIN_W:, 0])


def _build_lookup_kernel():
    info = plsc.get_sparse_core_info()
    nc, ns = info.num_cores, info.num_subcores
    nw = nc * ns
    rows_w = B // nw
    blocks_w = rows_w // L
    n_idx = N_SPARSE * rows_w

    mesh = plsc.VectorSubcoreMesh(core_axis_name="c", subcore_axis_name="s")

    @functools.partial(
        pl.kernel,
        out_type=jax.ShapeDtypeStruct((B,), jnp.float32),
        mesh=mesh,
        scratch_types=[
            pltpu.VMEM((N_COLS, rows_w), jnp.float32),
            pltpu.VMEM((n_idx,), jnp.int32),
            pltpu.VMEM((n_idx,), jnp.float32),
            pltpu.VMEM((N_DENSE * L,), jnp.float32),
            pltpu.VMEM((rows_w,), jnp.float32),
            pltpu.SemaphoreType.DMA,
        ],
    )
    def lookup_kernel(xt_hbm, flat_hbm, wb_hbm, out_hbm, xt_v, idx_v, gath_v,
                      wb_v, out_v, sem):
        wid = lax.axis_index("s") * nc + lax.axis_index("c")
        base_row = wid * rows_w

        pltpu.sync_copy(xt_hbm.at[:, pl.ds(base_row, rows_w)], xt_v)
        pltpu.sync_copy(wb_hbm, wb_v)

        def idx_body(t, _):
            f = t // blocks_w
            blk = t % blocks_w
            xv = xt_v[f, pl.ds(blk * L, L)]
            idx_v[pl.ds(t * L, L)] = xv.astype(jnp.int32) + f * FIELD_STRIDE
            return 0

        lax.fori_loop(0, N_SPARSE * blocks_w, idx_body, 0)

        pltpu.async_copy(flat_hbm.at[idx_v], gath_v, sem).wait()

        def blk_body(blk, _):
            def red_body(f, acc):
                g = gath_v[pl.ds(f * rows_w + blk * L, L)]
                return acc + g

            acc = lax.fori_loop(0, N_SPARSE, red_body,
                                jnp.zeros((L,), jnp.float32))

            def dense_body(d, dacc):
                xv = xt_v[N_SPARSE + d, pl.ds(blk * L, L)]
                wv = wb_v[pl.ds(d * L, L)]
                return dacc + xv * wv

            acc = lax.fori_loop(0, N_DENSE, dense_body, acc)
            out_v[pl.ds(blk * L, L)] = 1.0 / (1.0 + jnp.exp(-acc))
            return 0

        lax.fori_loop(0, blocks_w, blk_body, 0)

        pltpu.sync_copy(out_v, out_hbm.at[pl.ds(base_row, rows_w)])

    return lookup_kernel


def kernel(X, emb_tables, dense_weight):
    xt = X.T
    w_bcast = jnp.repeat(dense_weight.reshape(-1), L)
    flat = _tc_untile(emb_tables)
    out = _build_lookup_kernel()(xt, flat, w_bcast)
    return out[:, None]
